# Initial kernel scaffold; baseline (speedup 1.0000x reference)
import jax
import jax.numpy as jnp
from jax.experimental import pallas as pl


def kernel(source_cloud, target_cloud, source_mask, target_mask):
    raise NotImplementedError("write your pallas kernel here")


# speedup vs baseline: 1.1221x; 1.1221x over previous
import functools

import jax
import jax.numpy as jnp
from jax import lax
from jax.experimental import pallas as pl
from jax.experimental.pallas import tpu as pltpu
from jax.experimental.pallas import tpu_sc as plsc

NC = 2
NS = 16
L = 16
NW = NC * NS
NB = 8
NP = 2048
QPW = NP // NW
PADX = 1.0e5
BIG = 4.0e10


def _sqrt_vec(x):
    i = lax.bitcast_convert_type(x, jnp.int32)
    i = jnp.int32(0x5F3759DF) - lax.shift_right_logical(i, 1)
    y = lax.bitcast_convert_type(i, jnp.float32)
    for _ in range(3):
        y = y * (1.5 - 0.5 * x * y * y)
    return x * y


def _sc_chamfer(pts, masks):
    mesh = plsc.VectorSubcoreMesh(
        core_axis_name="c", subcore_axis_name="s",
        num_cores=NC, num_subcores=NS)

    @functools.partial(
        pl.kernel,
        out_type=jax.ShapeDtypeStruct((2, NB, NP), jnp.float32),
        mesh=mesh,
        scratch_types=[
            pltpu.VMEM((4, NB, NP), jnp.float32),
            pltpu.VMEM((L,), jnp.int32),
            pltpu.VMEM((QPW,), jnp.float32),
        ],
    )
    def k(pts_hbm, masks_hbm, out_hbm, pts_v, masks_v, res_v):
        wid = lax.axis_index("s") * NC + lax.axis_index("c")
        pltpu.sync_copy(pts_hbm, pts_v)
        pltpu.sync_copy(masks_hbm, masks_v)

        lane = lax.broadcasted_iota(jnp.int32, (L,), 0)
        mvec = masks_v[pl.ds(0, L)]

        for row, mbase in ((0, 0), (2, NB)):
            for b in range(NB):
                ln = mvec[mbase + b]

                def pad_body(g, _, row=row, b=b, ln=ln):
                    xs = pts_v[row, b, pl.ds(g * L, L)]
                    idx = lane + g * L
                    pts_v[row, b, pl.ds(g * L, L)] = jnp.where(
                        idx < ln, xs, PADX)
                    return 0

                lax.fori_loop(0, NP // L, pad_body, 0)

        for d in range(2):
            if d == 0:
                qxr, qyr, txr, tyr, qm, tm = 0, 1, 2, 3, 0, NB
            else:
                qxr, qyr, txr, tyr, qm, tm = 2, 3, 0, 1, NB, 0
            for b in range(NB):
                lent = mvec[tm + b]
                lenq = mvec[qm + b]
                ntg = (lent + (L - 1)) // L
                for qg in range(QPW // L):
                    qbase = wid * QPW + qg * L
                    qx = pts_v[qxr, b, pl.ds(qbase, L)]
                    qy = pts_v[qyr, b, pl.ds(qbase, L)]
                    init = jnp.full((L,), BIG, jnp.float32)

                    def body(tg, acc, b=b, txr=txr, tyr=tyr, qx=qx, qy=qy):
                        tvx = pts_v[txr, b, pl.ds(tg * L, L)]
                        tvy = pts_v[tyr, b, pl.ds(tg * L, L)]
                        for j in range(L):
                            dx = qx - tvx[j]
                            dy = qy - tvy[j]
                            acc = jnp.minimum(acc, dx * dx + dy * dy)
                        return acc

                    best = plsc.parallel_loop(
                        0, ntg, unroll=1, carry=init)(body)

                    eff_lenq = jnp.where(lent > 0, lenq, 0)
                    valid = (lane + qbase) < eff_lenq
                    res_v[pl.ds(qg * L, L)] = jnp.where(
                        valid, _sqrt_vec(best), 0.0)
                pltpu.sync_copy(
                    res_v, out_hbm.at[d, b, pl.ds(wid * QPW, QPW)])

    return k(pts, masks)


def kernel(source_cloud, target_cloud, source_mask, target_mask):
    pts = jnp.stack([
        source_cloud[:, :, 0], source_cloud[:, :, 1],
        target_cloud[:, :, 0], target_cloud[:, :, 1],
    ])
    masks = jnp.concatenate([source_mask, target_mask]).astype(jnp.int32)
    out = _sc_chamfer(pts, masks)
    return (out[0], out[1])


# speedup vs baseline: 1.4341x; 1.2780x over previous
import functools

import jax
import jax.numpy as jnp
from jax import lax
from jax.experimental import pallas as pl
from jax.experimental.pallas import tpu as pltpu
from jax.experimental.pallas import tpu_sc as plsc

NC = 2
NS = 16
L = 16
NW = NC * NS
NB = 8
NP = 2048
NG = NP // L
PADX = 1.0e5
BIG = 4.0e10


def _sqrt_vec(x):
    i = lax.bitcast_convert_type(x, jnp.int32)
    i = jnp.int32(0x5F3759DF) - lax.shift_right_logical(i, 1)
    y = lax.bitcast_convert_type(i, jnp.float32)
    for _ in range(3):
        y = y * (1.5 - 0.5 * x * y * y)
    return x * y


def _sc_chamfer(pts, masks):
    mesh = plsc.VectorSubcoreMesh(
        core_axis_name="c", subcore_axis_name="s",
        num_cores=NC, num_subcores=NS)

    @functools.partial(
        pl.kernel,
        out_type=jax.ShapeDtypeStruct((2, NB, NP), jnp.float32),
        mesh=mesh,
        scratch_types=[
            pltpu.VMEM((4, NB, NP), jnp.float32),
            pltpu.VMEM((L,), jnp.int32),
            pltpu.VMEM((4 * L,), jnp.float32),
        ],
    )
    def k(pts_hbm, masks_hbm, out_hbm, pts_v, masks_v, res_v):
        wid = lax.axis_index("s") * NC + lax.axis_index("c")
        pltpu.sync_copy(pts_hbm, pts_v)
        pltpu.sync_copy(masks_hbm, masks_v)

        lane = lax.broadcasted_iota(jnp.int32, (L,), 0)
        zeros = jnp.zeros((L,), jnp.float32)
        mvec = masks_v[pl.ds(0, L)]

        for row, mbase in ((0, 0), (2, NB)):
            for b in range(NB):
                ln = mvec[mbase + b]

                def pad_body(g, _, row=row, b=b, ln=ln):
                    xs = pts_v[row, b, pl.ds(g * L, L)]
                    idx = lane + g * L
                    pts_v[row, b, pl.ds(g * L, L)] = jnp.where(
                        idx < ln, xs, PADX)
                    return 0

                lax.fori_loop(0, NG, pad_body, 0)

        for d in range(2):
            if d == 0:
                qxr, qyr, txr, tyr, qm, tm = 0, 1, 2, 3, 0, NB
            else:
                qxr, qyr, txr, tyr, qm, tm = 2, 3, 0, 1, NB, 0
            for b in range(NB):
                lent = mvec[tm + b]
                lenq = mvec[qm + b]
                ntg = (lent + (L - 1)) // L
                ntgq = (lenq + (L - 1)) // L
                eff_lenq = jnp.where(lent > 0, lenq, 0)
                pmax = (jnp.where(ntgq > wid, 1, 0)
                        + jnp.where(ntgq > wid + 2 * NW, 1, 0))

                for j in range(4):
                    res_v[pl.ds(j * L, L)] = zeros

                def p_body(p, _, b=b, qxr=qxr, qyr=qyr, txr=txr, tyr=tyr,
                           ntg=ntg, eff_lenq=eff_lenq):
                    g0 = wid + 2 * NW * p
                    qb0 = g0 * L
                    qb1 = qb0 + NW * L
                    qx0 = pts_v[qxr, b, pl.ds(qb0, L)]
                    qy0 = pts_v[qyr, b, pl.ds(qb0, L)]
                    qx1 = pts_v[qxr, b, pl.ds(qb1, L)]
                    qy1 = pts_v[qyr, b, pl.ds(qb1, L)]
                    init = (jnp.full((L,), BIG, jnp.float32),
                            jnp.full((L,), BIG, jnp.float32))

                    def t_body(tg, acc, b=b, txr=txr, tyr=tyr,
                               qx0=qx0, qy0=qy0, qx1=qx1, qy1=qy1):
                        a0, a1 = acc
                        tvx = pts_v[txr, b, pl.ds(tg * L, L)]
                        tvy = pts_v[tyr, b, pl.ds(tg * L, L)]
                        for j in range(L):
                            bx = tvx[j]
                            by = tvy[j]
                            dx0 = qx0 - bx
                            dy0 = qy0 - by
                            a0 = jnp.minimum(a0, dx0 * dx0 + dy0 * dy0)
                            dx1 = qx1 - bx
                            dy1 = qy1 - by
                            a1 = jnp.minimum(a1, dx1 * dx1 + dy1 * dy1)
                        return a0, a1

                    a0, a1 = plsc.parallel_loop(
                        0, ntg, unroll=1, carry=init)(t_body)

                    v0 = (lane + qb0) < eff_lenq
                    v1 = (lane + qb1) < eff_lenq
                    res_v[pl.ds(2 * p * L, L)] = jnp.where(
                        v0, _sqrt_vec(a0), 0.0)
                    res_v[pl.ds((2 * p + 1) * L, L)] = jnp.where(
                        v1, _sqrt_vec(a1), 0.0)
                    return 0

                lax.fori_loop(0, pmax, p_body, 0)

                for j in range(4):
                    pltpu.sync_copy(
                        res_v.at[pl.ds(j * L, L)],
                        out_hbm.at[d, b, pl.ds((wid + NW * j) * L, L)])

    return k(pts, masks)


def kernel(source_cloud, target_cloud, source_mask, target_mask):
    pts = jnp.stack([
        source_cloud[:, :, 0], source_cloud[:, :, 1],
        target_cloud[:, :, 0], target_cloud[:, :, 1],
    ])
    masks = jnp.concatenate([source_mask, target_mask]).astype(jnp.int32)
    out = _sc_chamfer(pts, masks)
    return (out[0], out[1])


# speedup vs baseline: 1.5788x; 1.1009x over previous
import functools

import jax
import jax.numpy as jnp
from jax import lax
from jax.experimental import pallas as pl
from jax.experimental.pallas import tpu as pltpu
from jax.experimental.pallas import tpu_sc as plsc

NC = 2
NS = 16
L = 16
NW = NC * NS
NB = 8
NP = 2048
NG = NP // L
PADX = 1.0e5
BIG = 4.0e10


def _sqrt_vec(x):
    i = lax.bitcast_convert_type(x, jnp.int32)
    i = jnp.int32(0x5F3759DF) - lax.shift_right_logical(i, 1)
    y = lax.bitcast_convert_type(i, jnp.float32)
    for _ in range(3):
        y = y * (1.5 - 0.5 * x * y * y)
    return x * y


def _sc_chamfer(pts, masks):
    mesh = plsc.VectorSubcoreMesh(
        core_axis_name="c", subcore_axis_name="s",
        num_cores=NC, num_subcores=NS)

    @functools.partial(
        pl.kernel,
        out_type=(jax.ShapeDtypeStruct((NB, NP), jnp.float32),
                  jax.ShapeDtypeStruct((NB, NP), jnp.float32)),
        mesh=mesh,
        scratch_types=[
            pltpu.VMEM((4, NB, NP), jnp.float32),
            pltpu.VMEM((L,), jnp.int32),
            pltpu.VMEM((4 * L,), jnp.float32),
        ],
    )
    def k(pts_hbm, masks_hbm, fwd_hbm, bwd_hbm, pts_v, masks_v, res_v):
        wid = lax.axis_index("s") * NC + lax.axis_index("c")
        pltpu.sync_copy(pts_hbm, pts_v)
        pltpu.sync_copy(masks_hbm, masks_v)

        lane = lax.broadcasted_iota(jnp.int32, (L,), 0)
        zeros = jnp.zeros((L,), jnp.float32)
        mvec = masks_v[pl.ds(0, L)]

        for row, mbase in ((0, 0), (2, NB)):
            for b in range(NB):
                ln = mvec[mbase + b]
                g = ln // L
                xs = pts_v[row, b, pl.ds(g * L, L)]
                idx = lane + g * L
                pts_v[row, b, pl.ds(g * L, L)] = jnp.where(
                    idx < ln, xs, PADX)

        for d in range(2):
            if d == 0:
                qxr, qyr, txr, tyr, qm, tm = 0, 1, 2, 3, 0, NB
                out_hbm = fwd_hbm
            else:
                qxr, qyr, txr, tyr, qm, tm = 2, 3, 0, 1, NB, 0
                out_hbm = bwd_hbm
            for b in range(NB):
                lent = mvec[tm + b]
                lenq = mvec[qm + b]
                ntg = (lent + (L - 1)) // L
                ntgq = (lenq + (L - 1)) // L
                eff_lenq = jnp.where(lent > 0, lenq, 0)
                pmax = (jnp.where(ntgq > wid, 1, 0)
                        + jnp.where(ntgq > wid + 2 * NW, 1, 0))

                for j in range(4):
                    res_v[pl.ds(j * L, L)] = zeros

                def p_body(p, _, b=b, qxr=qxr, qyr=qyr, txr=txr, tyr=tyr,
                           ntg=ntg, eff_lenq=eff_lenq):
                    g0 = wid + 2 * NW * p
                    qb0 = g0 * L
                    qb1 = qb0 + NW * L
                    qx0 = pts_v[qxr, b, pl.ds(qb0, L)]
                    qy0 = pts_v[qyr, b, pl.ds(qb0, L)]
                    qx1 = pts_v[qxr, b, pl.ds(qb1, L)]
                    qy1 = pts_v[qyr, b, pl.ds(qb1, L)]
                    big = jnp.full((L,), BIG, jnp.float32)
                    init = (big, big, big, big)

                    def t_body(tg, acc, b=b, txr=txr, tyr=tyr,
                               qx0=qx0, qy0=qy0, qx1=qx1, qy1=qy1):
                        a0e, a0o, a1e, a1o = acc
                        tvx = pts_v[txr, b, pl.ds(tg * L, L)]
                        tvy = pts_v[tyr, b, pl.ds(tg * L, L)]
                        for j in range(L):
                            bx = tvx[j]
                            by = tvy[j]
                            dx0 = qx0 - bx
                            dy0 = qy0 - by
                            d0 = dx0 * dx0 + dy0 * dy0
                            dx1 = qx1 - bx
                            dy1 = qy1 - by
                            d1 = dx1 * dx1 + dy1 * dy1
                            if j % 2 == 0:
                                a0e = jnp.minimum(a0e, d0)
                                a1e = jnp.minimum(a1e, d1)
                            else:
                                a0o = jnp.minimum(a0o, d0)
                                a1o = jnp.minimum(a1o, d1)
                        return a0e, a0o, a1e, a1o

                    a0e, a0o, a1e, a1o = plsc.parallel_loop(
                        0, ntg, unroll=1, carry=init)(t_body)
                    a0 = jnp.minimum(a0e, a0o)
                    a1 = jnp.minimum(a1e, a1o)

                    v0 = (lane + qb0) < eff_lenq
                    v1 = (lane + qb1) < eff_lenq
                    res_v[pl.ds(2 * p * L, L)] = jnp.where(
                        v0, _sqrt_vec(a0), 0.0)
                    res_v[pl.ds((2 * p + 1) * L, L)] = jnp.where(
                        v1, _sqrt_vec(a1), 0.0)
                    return 0

                lax.fori_loop(0, pmax, p_body, 0)

                for j in range(4):
                    pltpu.sync_copy(
                        res_v.at[pl.ds(j * L, L)],
                        out_hbm.at[b, pl.ds((wid + NW * j) * L, L)])

    return k(pts, masks)


def kernel(source_cloud, target_cloud, source_mask, target_mask):
    pts = jnp.stack([
        source_cloud[:, :, 0], source_cloud[:, :, 1],
        target_cloud[:, :, 0], target_cloud[:, :, 1],
    ])
    masks = jnp.concatenate([source_mask, target_mask]).astype(jnp.int32)
    fwd, bwd = _sc_chamfer(pts, masks)
    return (fwd, bwd)


# speedup vs baseline: 1.6833x; 1.0662x over previous
import functools

import jax
import jax.numpy as jnp
from jax import lax
from jax.experimental import pallas as pl
from jax.experimental.pallas import tpu as pltpu
from jax.experimental.pallas import tpu_sc as plsc

NC = 2
NS = 16
L = 16
NW = NC * NS
NB = 8
NP = 2048
NG = NP // L
PADX = 1.0e5
BIG = 4.0e10


def _sqrt_vec(x):
    i = lax.bitcast_convert_type(x, jnp.int32)
    i = jnp.int32(0x5F3759DF) - lax.shift_right_logical(i, 1)
    y = lax.bitcast_convert_type(i, jnp.float32)
    for _ in range(3):
        y = y * (1.5 - 0.5 * x * y * y)
    return x * y


def _sc_chamfer(pts, masks):
    mesh = plsc.VectorSubcoreMesh(
        core_axis_name="c", subcore_axis_name="s",
        num_cores=NC, num_subcores=NS)

    @functools.partial(
        pl.kernel,
        out_type=(jax.ShapeDtypeStruct((NB, NP), jnp.float32),
                  jax.ShapeDtypeStruct((NB, NP), jnp.float32)),
        mesh=mesh,
        scratch_types=[
            pltpu.VMEM((4, NB, NP), jnp.float32),
            pltpu.VMEM((L,), jnp.int32),
            pltpu.VMEM((4 * L,), jnp.float32),
        ],
    )
    def k(pts_hbm, masks_hbm, fwd_hbm, bwd_hbm, pts_v, masks_v, res_v):
        wid = lax.axis_index("s") * NC + lax.axis_index("c")
        pltpu.sync_copy(pts_hbm, pts_v)
        pltpu.sync_copy(masks_hbm, masks_v)

        lane = lax.broadcasted_iota(jnp.int32, (L,), 0)
        zeros = jnp.zeros((L,), jnp.float32)
        mvec = masks_v[pl.ds(0, L)]

        for row, mbase in ((0, 0), (2, NB)):
            for b in range(NB):
                ln = mvec[mbase + b]
                g = ln // L
                xs = pts_v[row, b, pl.ds(g * L, L)]
                idx = lane + g * L
                pts_v[row, b, pl.ds(g * L, L)] = jnp.where(
                    idx < ln, xs, PADX)

        for d in range(2):
            if d == 0:
                qxr, qyr, txr, tyr, qm, tm = 0, 1, 2, 3, 0, NB
                out_hbm = fwd_hbm
            else:
                qxr, qyr, txr, tyr, qm, tm = 2, 3, 0, 1, NB, 0
                out_hbm = bwd_hbm
            for b in range(NB):
                lent = mvec[tm + b]
                lenq = mvec[qm + b]
                ntg = (lent + (L - 1)) // L
                ntgq = (lenq + (L - 1)) // L
                eff_lenq = jnp.where(lent > 0, lenq, 0)
                pmax = (jnp.where(ntgq > wid, 1, 0)
                        + jnp.where(ntgq > wid + 2 * NW, 1, 0))

                for j in range(4):
                    res_v[pl.ds(j * L, L)] = zeros

                def p_body(p, _, b=b, qxr=qxr, qyr=qyr, txr=txr, tyr=tyr,
                           ntg=ntg, eff_lenq=eff_lenq):
                    g0 = wid + 2 * NW * p
                    qb0 = g0 * L
                    qb1 = qb0 + NW * L
                    qx0 = pts_v[qxr, b, pl.ds(qb0, L)]
                    qy0 = pts_v[qyr, b, pl.ds(qb0, L)]
                    qx1 = pts_v[qxr, b, pl.ds(qb1, L)]
                    qy1 = pts_v[qyr, b, pl.ds(qb1, L)]
                    big = jnp.full((L,), BIG, jnp.float32)
                    init = (big, big, big, big)

                    def t_body(tg, acc, b=b, txr=txr, tyr=tyr,
                               qx0=qx0, qy0=qy0, qx1=qx1, qy1=qy1):
                        a0e, a0o, a1e, a1o = acc
                        tvx = pts_v[txr, b, pl.ds(tg * L, L)]
                        tvy = pts_v[tyr, b, pl.ds(tg * L, L)]
                        for c in range(2):
                            zd = a0e * 0.0
                            qx0c = qx0 + zd
                            qx1c = qx1 + zd
                            for j in range(8 * c, 8 * c + 8):
                                bx = tvx[j]
                                by = tvy[j]
                                dx0 = qx0c - bx
                                dy0 = qy0 - by
                                d0 = dx0 * dx0 + dy0 * dy0
                                dx1 = qx1c - bx
                                dy1 = qy1 - by
                                d1 = dx1 * dx1 + dy1 * dy1
                                if j % 2 == 0:
                                    a0e = jnp.minimum(a0e, d0)
                                    a1e = jnp.minimum(a1e, d1)
                                else:
                                    a0o = jnp.minimum(a0o, d0)
                                    a1o = jnp.minimum(a1o, d1)
                        return a0e, a0o, a1e, a1o

                    a0e, a0o, a1e, a1o = plsc.parallel_loop(
                        0, ntg, unroll=1, carry=init)(t_body)
                    a0 = jnp.minimum(a0e, a0o)
                    a1 = jnp.minimum(a1e, a1o)

                    v0 = (lane + qb0) < eff_lenq
                    v1 = (lane + qb1) < eff_lenq
                    res_v[pl.ds(2 * p * L, L)] = jnp.where(
                        v0, _sqrt_vec(a0), 0.0)
                    res_v[pl.ds((2 * p + 1) * L, L)] = jnp.where(
                        v1, _sqrt_vec(a1), 0.0)
                    return 0

                lax.fori_loop(0, pmax, p_body, 0)

                for j in range(4):
                    pltpu.sync_copy(
                        res_v.at[pl.ds(j * L, L)],
                        out_hbm.at[b, pl.ds((wid + NW * j) * L, L)])

    return k(pts, masks)


def kernel(source_cloud, target_cloud, source_mask, target_mask):
    pts = jnp.stack([
        source_cloud[:, :, 0], source_cloud[:, :, 1],
        target_cloud[:, :, 0], target_cloud[:, :, 1],
    ])
    masks = jnp.concatenate([source_mask, target_mask]).astype(jnp.int32)
    fwd, bwd = _sc_chamfer(pts, masks)
    return (fwd, bwd)


# speedup vs baseline: 1.6974x; 1.0084x over previous
import functools

import jax
import jax.numpy as jnp
from jax import lax
from jax.experimental import pallas as pl
from jax.experimental.pallas import tpu as pltpu
from jax.experimental.pallas import tpu_sc as plsc

NC = 2
NS = 16
L = 16
NW = NC * NS
NB = 8
NP = 2048
NG = NP // L
PADX = 1.0e5
BIG = 4.0e10


def _sqrt_vec(x):
    i = lax.bitcast_convert_type(x, jnp.int32)
    i = jnp.int32(0x5F3759DF) - lax.shift_right_logical(i, 1)
    y = lax.bitcast_convert_type(i, jnp.float32)
    for _ in range(3):
        y = y * (1.5 - 0.5 * x * y * y)
    return x * y


def _sc_chamfer(pts, masks):
    mesh = plsc.VectorSubcoreMesh(
        core_axis_name="c", subcore_axis_name="s",
        num_cores=NC, num_subcores=NS)

    @functools.partial(
        pl.kernel,
        out_type=(jax.ShapeDtypeStruct((NB, NP), jnp.float32),
                  jax.ShapeDtypeStruct((NB, NP), jnp.float32)),
        mesh=mesh,
        scratch_types=[
            pltpu.VMEM((4, NB, NP), jnp.float32),
            pltpu.VMEM((L,), jnp.int32),
            pltpu.VMEM((4 * L,), jnp.float32),
        ],
    )
    def k(pts_hbm, masks_hbm, fwd_hbm, bwd_hbm, pts_v, masks_v, res_v):
        wid = lax.axis_index("s") * NC + lax.axis_index("c")
        pltpu.sync_copy(pts_hbm, pts_v)
        pltpu.sync_copy(masks_hbm, masks_v)

        lane = lax.broadcasted_iota(jnp.int32, (L,), 0)
        zeros = jnp.zeros((L,), jnp.float32)
        mvec = masks_v[pl.ds(0, L)]

        for row, mbase in ((0, 0), (2, NB)):
            for b in range(NB):
                ln = mvec[mbase + b]
                g = ln // L
                xs = pts_v[row, b, pl.ds(g * L, L)]
                idx = lane + g * L
                pts_v[row, b, pl.ds(g * L, L)] = jnp.where(
                    idx < ln, xs, PADX)

        for d in range(2):
            if d == 0:
                qxr, qyr, txr, tyr, qm, tm = 0, 1, 2, 3, 0, NB
                out_hbm = fwd_hbm
            else:
                qxr, qyr, txr, tyr, qm, tm = 2, 3, 0, 1, NB, 0
                out_hbm = bwd_hbm
            for b in range(NB):
                lent = mvec[tm + b]
                lenq = mvec[qm + b]
                ntg = (lent + (L - 1)) // L
                ntgq = (lenq + (L - 1)) // L
                eff_lenq = jnp.where(lent > 0, lenq, 0)
                pmax = (jnp.where(ntgq > wid, 1, 0)
                        + jnp.where(ntgq > wid + 2 * NW, 1, 0))

                for j in range(4):
                    res_v[pl.ds(j * L, L)] = zeros

                def p_body(p, _, b=b, qxr=qxr, qyr=qyr, txr=txr, tyr=tyr,
                           ntg=ntg, eff_lenq=eff_lenq):
                    g0 = wid + 2 * NW * p
                    qb0 = g0 * L
                    qb1 = qb0 + NW * L
                    qx0 = pts_v[qxr, b, pl.ds(qb0, L)]
                    qy0 = pts_v[qyr, b, pl.ds(qb0, L)]
                    qx1 = pts_v[qxr, b, pl.ds(qb1, L)]
                    qy1 = pts_v[qyr, b, pl.ds(qb1, L)]
                    big = jnp.full((L,), BIG, jnp.float32)
                    init = (big, big, big, big)

                    def t_body(tg, acc, b=b, txr=txr, tyr=tyr,
                               qx0=qx0, qy0=qy0, qx1=qx1, qy1=qy1):
                        a0e, a0o, a1e, a1o = acc
                        tvx = pts_v[txr, b, pl.ds(tg * L, L)]
                        tvy = pts_v[tyr, b, pl.ds(tg * L, L)]
                        for c in range(1):
                            zd = a0e * 0.0
                            qx0c = qx0 + zd
                            qx1c = qx1 + zd
                            for j in range(16 * c, 16 * c + 16):
                                bx = tvx[j]
                                by = tvy[j]
                                dx0 = qx0c - bx
                                dy0 = qy0 - by
                                d0 = dx0 * dx0 + dy0 * dy0
                                dx1 = qx1c - bx
                                dy1 = qy1 - by
                                d1 = dx1 * dx1 + dy1 * dy1
                                if j % 2 == 0:
                                    a0e = jnp.minimum(a0e, d0)
                                    a1e = jnp.minimum(a1e, d1)
                                else:
                                    a0o = jnp.minimum(a0o, d0)
                                    a1o = jnp.minimum(a1o, d1)
                        return a0e, a0o, a1e, a1o

                    a0e, a0o, a1e, a1o = plsc.parallel_loop(
                        0, ntg, unroll=1, carry=init)(t_body)
                    a0 = jnp.minimum(a0e, a0o)
                    a1 = jnp.minimum(a1e, a1o)

                    v0 = (lane + qb0) < eff_lenq
                    v1 = (lane + qb1) < eff_lenq
                    res_v[pl.ds(2 * p * L, L)] = jnp.where(
                        v0, _sqrt_vec(a0), 0.0)
                    res_v[pl.ds((2 * p + 1) * L, L)] = jnp.where(
                        v1, _sqrt_vec(a1), 0.0)
                    return 0

                lax.fori_loop(0, pmax, p_body, 0)

                for j in range(4):
                    pltpu.sync_copy(
                        res_v.at[pl.ds(j * L, L)],
                        out_hbm.at[b, pl.ds((wid + NW * j) * L, L)])

    return k(pts, masks)


def kernel(source_cloud, target_cloud, source_mask, target_mask):
    pts = jnp.stack([
        source_cloud[:, :, 0], source_cloud[:, :, 1],
        target_cloud[:, :, 0], target_cloud[:, :, 1],
    ])
    masks = jnp.concatenate([source_mask, target_mask]).astype(jnp.int32)
    fwd, bwd = _sc_chamfer(pts, masks)
    return (fwd, bwd)
